# baseline (speedup 1.0000x reference)
import functools

import jax
import jax.numpy as jnp
from jax import lax
from jax.experimental import pallas as pl
from jax.experimental.pallas import tpu as pltpu
from jax.experimental.pallas import tpu_sc as plsc

D_MODEL = 1024
MAX_REL = 128


def _make_pos_embed_sc(seq_len: int):
    B = 2 * seq_len - 1
    B_pad = 2 * seq_len
    info = plsc.get_sparse_core_info()
    NC, NS, L = info.num_cores, info.num_subcores, info.num_lanes
    NW = NC * NS
    b_per_w = B_pad // NW
    CHUNK = 64
    n_chunks = b_per_w // CHUNK
    dist = seq_len - 1

    lo, hi = dist - MAX_REL, dist + MAX_REL
    for w in range(NW):
        has0 = any(w * b_per_w + c * CHUNK + CHUNK - 1 < lo
                   for c in range(n_chunks))
        has1 = any(w * b_per_w + c * CHUNK > hi for c in range(n_chunks))
        assert not (has0 and has1), "worker spans both clip regions"

    mesh = plsc.VectorSubcoreMesh(core_axis_name="c", subcore_axis_name="s")

    @functools.partial(
        pl.kernel,
        mesh=mesh,
        out_type=jax.ShapeDtypeStruct((B, D_MODEL), jnp.float32),
        scratch_types=[
            pltpu.VMEM((CHUNK,), jnp.int32),
            pltpu.VMEM((L,), jnp.int32),
            pltpu.VMEM((32, D_MODEL), jnp.float32),
            pltpu.VMEM((CHUNK, D_MODEL), jnp.float32),
            pltpu.SemaphoreType.DMA,
            pltpu.SemaphoreType.DMA,
        ],
    )
    def pos_embed_kernel(amp_hbm, pe_hbm, out_hbm, gidx_v, tidx_v, rep_v,
                         big_v, gsem, osem):
        wid = lax.axis_index("s") * NC + lax.axis_index("c")
        base_w = wid * b_per_w

        def table_idx(r):
            return jnp.clip(r - dist, -MAX_REL, MAX_REL) + MAX_REL

        i_first = table_idx(base_w)
        chunk0_const = i_first == table_idx(base_w + CHUNK - 1)
        amp_row = jnp.where(chunk0_const, i_first,
                            table_idx(base_w + b_per_w - 1))
        amp_off = jnp.where(amp_row == 0, 0, 32)

        pltpu.sync_copy(amp_hbm.at[pl.ds(amp_off, 32)], rep_v)

        for c in range(n_chunks):
            cbase = base_w + c * CHUNK
            c_const = table_idx(cbase) == table_idx(cbase + CHUNK - 1)
            is_tail = cbase == B_pad - CHUNK

            @pl.when(c_const & jnp.logical_not(is_tail))
            def _():
                cps = [
                    pltpu.async_copy(
                        rep_v, out_hbm.at[pl.ds(cbase + 32 * k, 32)], osem)
                    for k in range(CHUNK // 32)
                ]
                for cp in cps:
                    cp.wait()

            @pl.when(c_const & is_tail)
            def _():
                cps = [pltpu.async_copy(
                    rep_v, out_hbm.at[pl.ds(cbase, 32)], osem)]
                cps.append(pltpu.async_copy(
                    rep_v.at[pl.ds(0, 16)],
                    out_hbm.at[pl.ds(cbase + 32, 16)], osem))
                cps.append(pltpu.async_copy(
                    rep_v.at[pl.ds(0, 8)],
                    out_hbm.at[pl.ds(cbase + 48, 8)], osem))
                rr = lax.iota(jnp.int32, L) + (B_pad - L)
                tidx_v[...] = jnp.minimum(rr, B - 1)
                cps.append(pltpu.async_copy(
                    rep_v.at[pl.ds(0, L)], out_hbm.at[tidx_v], osem))
                for cp in cps:
                    cp.wait()

            @pl.when(jnp.logical_not(c_const))
            def _():
                for j in range(CHUNK // L):
                    r = lax.iota(jnp.int32, L) + (cbase + j * L)
                    gidx_v[pl.ds(j * L, L)] = table_idx(r)
                pltpu.async_copy(pe_hbm.at[gidx_v], big_v, gsem).wait()
                pltpu.sync_copy(big_v, out_hbm.at[pl.ds(cbase, CHUNK)])

    return pos_embed_kernel


def _x_copy_tc(x):
    b = x.shape[0]

    def copy_body(x_hbm, o_hbm, sem):
        cps = [pltpu.async_copy(x_hbm.at[i], o_hbm.at[i], sem)
               for i in range(b)]
        for cp in cps:
            cp.wait()

    return pl.pallas_call(
        copy_body,
        in_specs=[pl.BlockSpec(memory_space=pltpu.MemorySpace.HBM)],
        out_specs=pl.BlockSpec(memory_space=pltpu.MemorySpace.HBM),
        out_shape=jax.ShapeDtypeStruct(x.shape, x.dtype),
        scratch_shapes=[pltpu.SemaphoreType.DMA],
    )(x)


def kernel(x, pe):
    seq_len = x.shape[1]
    amp = jnp.concatenate([
        jnp.broadcast_to(pe[0], (32, pe.shape[1])),
        jnp.broadcast_to(pe[2 * MAX_REL], (32, pe.shape[1])),
    ])
    pos_embed = _make_pos_embed_sc(seq_len)(amp, pe)
    x_out = _x_copy_tc(x)
    return (x_out, pos_embed)


# speedup vs baseline: 27.8570x; 27.8570x over previous
import functools

import jax
import jax.numpy as jnp
from jax import lax
from jax.experimental import pallas as pl
from jax.experimental.pallas import tpu as pltpu
from jax.experimental.pallas import tpu_sc as plsc

D_MODEL = 1024
MAX_REL = 128


def _make_pos_embed_sc(seq_len: int):
    B = 2 * seq_len - 1
    B_pad = 2 * seq_len
    info = plsc.get_sparse_core_info()
    NC, NS, L = info.num_cores, info.num_subcores, info.num_lanes
    NW = NC * NS
    b_per_w = B_pad // NW
    CHUNK = 64
    n_chunks = b_per_w // CHUNK
    dist = seq_len - 1

    lo, hi = dist - MAX_REL, dist + MAX_REL
    for w in range(NW):
        has0 = any(w * b_per_w + c * CHUNK + CHUNK - 1 < lo
                   for c in range(n_chunks))
        has1 = any(w * b_per_w + c * CHUNK > hi for c in range(n_chunks))
        assert not (has0 and has1), "worker spans both clip regions"

    mesh = plsc.VectorSubcoreMesh(core_axis_name="c", subcore_axis_name="s")

    @functools.partial(
        pl.kernel,
        mesh=mesh,
        out_type=jax.ShapeDtypeStruct((B, D_MODEL), jnp.float32),
        scratch_types=[
            pltpu.VMEM((CHUNK,), jnp.int32),
            pltpu.VMEM((L,), jnp.int32),
            pltpu.VMEM((32, D_MODEL), jnp.float32),
            pltpu.VMEM((CHUNK, D_MODEL), jnp.float32),
            pltpu.SemaphoreType.DMA,
            pltpu.SemaphoreType.DMA,
        ],
    )
    def pos_embed_kernel(amp_hbm, pe_hbm, out_hbm, gidx_v, tidx_v, rep_v,
                         big_v, gsem, osem):
        wid = lax.axis_index("s") * NC + lax.axis_index("c")
        base_w = wid * b_per_w

        def table_idx(r):
            return jnp.clip(r - dist, -MAX_REL, MAX_REL) + MAX_REL

        i_first = table_idx(base_w)
        chunk0_const = i_first == table_idx(base_w + CHUNK - 1)
        amp_row = jnp.where(chunk0_const, i_first,
                            table_idx(base_w + b_per_w - 1))
        amp_off = jnp.where(amp_row == 0, 0, 32)

        pltpu.sync_copy(amp_hbm.at[pl.ds(amp_off, 32)], rep_v)

        for c in range(n_chunks):
            cbase = base_w + c * CHUNK
            c_const = table_idx(cbase) == table_idx(cbase + CHUNK - 1)
            is_tail = cbase == B_pad - CHUNK

            @pl.when(c_const & jnp.logical_not(is_tail))
            def _():
                cps = [
                    pltpu.async_copy(
                        rep_v, out_hbm.at[pl.ds(cbase + 32 * k, 32)], osem)
                    for k in range(CHUNK // 32)
                ]
                for cp in cps:
                    cp.wait()

            @pl.when(c_const & is_tail)
            def _():
                cps = [pltpu.async_copy(
                    rep_v, out_hbm.at[pl.ds(cbase, 32)], osem)]
                cps.append(pltpu.async_copy(
                    rep_v.at[pl.ds(0, 16)],
                    out_hbm.at[pl.ds(cbase + 32, 16)], osem))
                cps.append(pltpu.async_copy(
                    rep_v.at[pl.ds(0, 8)],
                    out_hbm.at[pl.ds(cbase + 48, 8)], osem))
                rr = lax.iota(jnp.int32, L) + (B_pad - L)
                tidx_v[...] = jnp.minimum(rr, B - 1)
                cps.append(pltpu.async_copy(
                    rep_v.at[pl.ds(0, L)], out_hbm.at[tidx_v], osem))
                for cp in cps:
                    cp.wait()

            @pl.when(jnp.logical_not(c_const))
            def _():
                for j in range(CHUNK // L):
                    r = lax.iota(jnp.int32, L) + (cbase + j * L)
                    gidx_v[pl.ds(j * L, L)] = table_idx(r)
                pltpu.async_copy(pe_hbm.at[gidx_v], big_v, gsem).wait()
                pltpu.sync_copy(big_v, out_hbm.at[pl.ds(cbase, CHUNK)])

    return pos_embed_kernel


def _x_copy_tc(x):
    b, s, d = x.shape
    x2 = x.reshape(b * s, d)
    blk = 2048
    out = pl.pallas_call(
        lambda x_ref, o_ref: o_ref.__setitem__((...,), x_ref[...]),
        grid=(b * s // blk,),
        in_specs=[pl.BlockSpec((blk, d), lambda i: (i, 0))],
        out_specs=pl.BlockSpec((blk, d), lambda i: (i, 0)),
        out_shape=jax.ShapeDtypeStruct(x2.shape, x2.dtype),
    )(x2)
    return out.reshape(x.shape)


def kernel(x, pe):
    seq_len = x.shape[1]
    amp = jnp.concatenate([
        jnp.broadcast_to(pe[0], (32, pe.shape[1])),
        jnp.broadcast_to(pe[2 * MAX_REL], (32, pe.shape[1])),
    ])
    pos_embed = _make_pos_embed_sc(seq_len)(amp, pe)
    x_out = _x_copy_tc(x)
    return (x_out, pos_embed)


# speedup vs baseline: 28.2695x; 1.0148x over previous
import functools

import jax
import jax.numpy as jnp
from jax import lax
from jax.experimental import pallas as pl
from jax.experimental.pallas import tpu as pltpu
from jax.experimental.pallas import tpu_sc as plsc

D_MODEL = 1024
MAX_REL = 128


def _make_pos_embed_sc(seq_len: int):
    B = 2 * seq_len - 1
    B_pad = 2 * seq_len
    info = plsc.get_sparse_core_info()
    NC, NS, L = info.num_cores, info.num_subcores, info.num_lanes
    NW = NC * NS
    b_per_w = B_pad // NW
    CHUNK = 64
    n_chunks = b_per_w // CHUNK
    dist = seq_len - 1

    lo, hi = dist - MAX_REL, dist + MAX_REL
    for w in range(NW):
        has0 = any(w * b_per_w + c * CHUNK + CHUNK - 1 < lo
                   for c in range(n_chunks))
        has1 = any(w * b_per_w + c * CHUNK > hi for c in range(n_chunks))
        assert not (has0 and has1), "worker spans both clip regions"

    mesh = plsc.VectorSubcoreMesh(core_axis_name="c", subcore_axis_name="s")

    @functools.partial(
        pl.kernel,
        mesh=mesh,
        out_type=jax.ShapeDtypeStruct((B, D_MODEL), jnp.float32),
        scratch_types=[
            pltpu.VMEM((CHUNK,), jnp.int32),
            pltpu.VMEM((L,), jnp.int32),
            pltpu.VMEM((32, D_MODEL), jnp.float32),
            pltpu.VMEM((CHUNK, D_MODEL), jnp.float32),
            pltpu.SemaphoreType.DMA,
            pltpu.SemaphoreType.DMA,
        ],
    )
    def pos_embed_kernel(amp_hbm, pe_hbm, out_hbm, gidx_v, tidx_v, rep_v,
                         big_v, gsem, osem):
        wid = lax.axis_index("s") * NC + lax.axis_index("c")
        base_w = wid * b_per_w

        def table_idx(r):
            return jnp.clip(r - dist, -MAX_REL, MAX_REL) + MAX_REL

        i_first = table_idx(base_w)
        chunk0_const = i_first == table_idx(base_w + CHUNK - 1)
        amp_row = jnp.where(chunk0_const, i_first,
                            table_idx(base_w + b_per_w - 1))
        amp_off = jnp.where(amp_row == 0, 0, 32)

        pltpu.sync_copy(amp_hbm.at[pl.ds(amp_off, 32)], rep_v)

        for c in range(n_chunks):
            cbase = base_w + c * CHUNK
            c_const = table_idx(cbase) == table_idx(cbase + CHUNK - 1)
            is_tail = cbase == B_pad - CHUNK

            @pl.when(c_const & jnp.logical_not(is_tail))
            def _():
                cps = [
                    pltpu.async_copy(
                        rep_v, out_hbm.at[pl.ds(cbase + 32 * k, 32)], osem)
                    for k in range(CHUNK // 32)
                ]
                for cp in cps:
                    cp.wait()

            @pl.when(c_const & is_tail)
            def _():
                cps = [pltpu.async_copy(
                    rep_v, out_hbm.at[pl.ds(cbase, 32)], osem)]
                cps.append(pltpu.async_copy(
                    rep_v.at[pl.ds(0, 16)],
                    out_hbm.at[pl.ds(cbase + 32, 16)], osem))
                cps.append(pltpu.async_copy(
                    rep_v.at[pl.ds(0, 8)],
                    out_hbm.at[pl.ds(cbase + 48, 8)], osem))
                rr = lax.iota(jnp.int32, L) + (B_pad - L)
                tidx_v[...] = jnp.minimum(rr, B - 1)
                cps.append(pltpu.async_copy(
                    rep_v.at[pl.ds(0, L)], out_hbm.at[tidx_v], osem))
                for cp in cps:
                    cp.wait()

            @pl.when(jnp.logical_not(c_const))
            def _():
                for j in range(CHUNK // L):
                    r = lax.iota(jnp.int32, L) + (cbase + j * L)
                    gidx_v[pl.ds(j * L, L)] = table_idx(r)
                pltpu.async_copy(pe_hbm.at[gidx_v], big_v, gsem).wait()
                pltpu.sync_copy(big_v, out_hbm.at[pl.ds(cbase, CHUNK)])

    return pos_embed_kernel


def _x_copy_tc(x):
    b, s, d = x.shape
    x2 = x.reshape(b * s, d)
    rows = b * s
    blk = 2048
    n = rows // blk
    depth = 4

    def copy_body(x_hbm, o_hbm, buf, in_sem, out_sem):
        def in_cp(c, slot):
            return pltpu.make_async_copy(
                x_hbm.at[pl.ds(c * blk, blk)], buf.at[slot], in_sem.at[slot])

        def out_cp(c, slot):
            return pltpu.make_async_copy(
                buf.at[slot], o_hbm.at[pl.ds(c * blk, blk)], out_sem.at[slot])

        for c in range(depth):
            in_cp(c, c).start()
        for c in range(n):
            slot = c % depth
            in_cp(c, slot).wait()
            out_cp(c, slot).start()
            if c + depth < n:
                out_cp(c, slot).wait()
                in_cp(c + depth, slot).start()
        for c in range(n - depth, n):
            out_cp(c, c % depth).wait()

    out = pl.pallas_call(
        copy_body,
        in_specs=[pl.BlockSpec(memory_space=pltpu.MemorySpace.HBM)],
        out_specs=pl.BlockSpec(memory_space=pltpu.MemorySpace.HBM),
        out_shape=jax.ShapeDtypeStruct(x2.shape, x2.dtype),
        scratch_shapes=[
            pltpu.VMEM((depth, blk, d), x.dtype),
            pltpu.SemaphoreType.DMA((depth,)),
            pltpu.SemaphoreType.DMA((depth,)),
        ],
    )(x2)
    return out.reshape(x.shape)


def kernel(x, pe):
    seq_len = x.shape[1]
    amp = jnp.concatenate([
        jnp.broadcast_to(pe[0], (32, pe.shape[1])),
        jnp.broadcast_to(pe[2 * MAX_REL], (32, pe.shape[1])),
    ])
    pos_embed = _make_pos_embed_sc(seq_len)(amp, pe)
    x_out = _x_copy_tc(x)
    return (x_out, pos_embed)
